# baseline (speedup 1.0000x reference)
import jax
import jax.numpy as jnp
from jax import lax
from jax.experimental import pallas as pl
from jax.experimental.pallas import tpu as pltpu
from jax.experimental.pallas import tpu_sc as plsc

_K = 5
_L = 16
_NCAND = _K * _L


def _tc_d_body(fc_ref, fs_ref, d_ref):
    f32 = jnp.float32
    fc = fc_ref[...]
    fs = fs_ref[...]
    m, F = fc.shape
    eps = f32(1e-8)

    hi = jax.lax.Precision.HIGHEST
    ones_row_F = jnp.ones((1, F), f32)
    nc_col = jnp.sum(fc * fc, axis=1, keepdims=True)
    nc_row = jax.lax.dot_general(ones_row_F, fc * fc, (((1,), (1,)), ((), ())),
                                 precision=hi, preferred_element_type=f32)
    ns_row = jax.lax.dot_general(ones_row_F, fs * fs, (((1,), (1,)), ((), ())),
                                 precision=hi, preferred_element_type=f32)

    def ncc(x, x_norm_row):
        g = jax.lax.dot_general(fc, x, (((1,), (1,)), ((), ())),
                                preferred_element_type=f32)
        return (g + eps) / (jnp.sqrt(nc_col * x_norm_row) + eps)

    d_ref[pl.ds(0, m), :] = ncc(fc, nc_row)
    d_ref[pl.ds(m, m), :] = ncc(fs, ns_row)


def _sc_topcand_body(d_hbm, cv_hbm, buf0, buf1, oval, sem0, sem1):
    info = plsc.get_sparse_core_info()
    nw = info.num_cores * info.num_subcores
    wid = lax.axis_index("s") * info.num_cores + lax.axis_index("c")
    ch, n = buf0.shape
    rows = cv_hbm.shape[0]
    rpw = rows // nw
    n_chunks = rpw // ch
    base = wid * rpw
    bufs = [buf0, buf1]
    sems = [sem0, sem1]
    neg_inf = jnp.full((_L,), -jnp.inf, jnp.float32)

    h = pltpu.async_copy(d_hbm.at[pl.ds(base, ch)], bufs[0], sems[0])
    for c in range(n_chunks):
        nxt = None
        if c + 1 < n_chunks:
            nxt = pltpu.async_copy(
                d_hbm.at[pl.ds(base + (c + 1) * ch, ch)],
                bufs[(c + 1) % 2], sems[(c + 1) % 2])
        h.wait()
        buf = bufs[c % 2]

        def row_body(r, _, buf=buf):
            v = [neg_inf] * _K
            for blk in range(n // _L):
                x = buf[r, pl.ds(blk * _L, _L)]
                for t in range(_K):
                    lo = jnp.minimum(x, v[t])
                    v[t] = jnp.maximum(x, v[t])
                    x = lo
            for t in range(_K):
                oval[r, pl.ds(t * _L, _L)] = v[t]
            return 0

        lax.fori_loop(0, ch, row_body, 0)
        pltpu.sync_copy(oval, cv_hbm.at[pl.ds(base + c * ch, ch)])
        h = nxt


def _sc_topcand(d_flat):
    rows, n = d_flat.shape
    ch = 16
    mesh = plsc.VectorSubcoreMesh(core_axis_name="c", subcore_axis_name="s")
    fn = pl.kernel(
        _sc_topcand_body,
        out_type=jax.ShapeDtypeStruct((rows, _NCAND), jnp.float32),
        mesh=mesh,
        scratch_types=[
            pltpu.VMEM((ch, n), jnp.float32),
            pltpu.VMEM((ch, n), jnp.float32),
            pltpu.VMEM((ch, _NCAND), jnp.float32),
            pltpu.SemaphoreType.DMA,
            pltpu.SemaphoreType.DMA,
        ],
    )
    return fn(d_flat)


def _tc_agg_body(fc_ref, fs_ref, d_ref, cv_ref, W1_ref, b1_ref, W2_ref,
                 b2_ref, out_ref):
    f32 = jnp.float32
    fc = fc_ref[...]
    fs = fs_ref[...]
    m, F = fc.shape
    n = fs.shape[0]
    D = d_ref[...]
    val = cv_ref[...]

    lane80 = lax.broadcasted_iota(jnp.int32, val.shape, 1)
    big = jnp.int32(1 << 30)
    mx = jnp.max(val, axis=1, keepdims=True)
    for _ in range(_K - 1):
        am = jnp.min(jnp.where(val == mx, lane80, big), axis=1, keepdims=True)
        val = jnp.where(lane80 == am, -jnp.inf, val)
        mx = jnp.max(val, axis=1, keepdims=True)
    T = mx

    sel_gt = (D > T).astype(f32)
    eq = (D == T).astype(f32)
    ones_col_n = jnp.ones((n, 1), f32)

    def mm(a, b):
        return jax.lax.dot_general(a, b, (((1,), (0,)), ((), ())),
                                   preferred_element_type=f32)

    g = mm(sel_gt, ones_col_n)
    r_iota = lax.broadcasted_iota(jnp.int32, (n, n), 0)
    c_iota = lax.broadcasted_iota(jnp.int32, (n, n), 1)
    U = (r_iota <= c_iota).astype(f32)
    csum = mm(eq, U)
    S = sel_gt + eq * (csum <= (jnp.float32(_K) - g)).astype(f32)

    S2 = S[:m, :]
    S1 = S[m:, :]

    ones_row_m = jnp.ones((1, m), f32)
    cnt2 = jax.lax.dot_general(ones_row_m, S2, (((1,), (0,)), ((), ())),
                               preferred_element_type=f32)
    cnt1 = jax.lax.dot_general(ones_row_m, S1, (((1,), (0,)), ((), ())),
                               preferred_element_type=f32)
    S2w = S2 * jax.lax.rsqrt(jnp.maximum(cnt2, 1.0))
    S1w = S1 * jax.lax.rsqrt(jnp.maximum(cnt1, 1.0))

    c_in = f32((2.0 * _K) ** -0.5)
    W1 = W1_ref[...]
    W2 = W2_ref[...]
    b1 = b1_ref[...]
    b2 = b2_ref[...]

    agg1 = (mm(S2w, fc) + mm(S1w, fs)) * c_in
    h1 = jnp.maximum(mm(agg1, W1) + b1, 0.0)
    h1s = jnp.maximum(b1, 0.0)

    rs1 = mm(S1w, ones_col_n)
    agg2 = (mm(S2w, h1) + rs1 * h1s) * c_in
    out_ref[...] = mm(agg2, W2) + b2


def kernel(ys, yc, W1, b1, W2, b2):
    B, N1, N2, F = ys.shape
    _, C, P, _, M1, M2 = yc.shape
    n = N1 * N2
    m = M1 * M2
    fs_all = ys.reshape(B, n, F)
    fc_all = jnp.transpose(yc, (0, 4, 5, 1, 2, 3)).reshape(B, m, F)
    b1r = b1.reshape(1, F)
    b2r = b2.reshape(1, F)

    tc1 = pl.pallas_call(
        _tc_d_body,
        out_shape=jax.ShapeDtypeStruct((2 * m, n), jnp.float32),
    )
    tc2 = pl.pallas_call(
        _tc_agg_body,
        out_shape=jax.ShapeDtypeStruct((m, F), jnp.float32),
    )

    outs = []
    for b in range(B):
        fc = fc_all[b]
        fs = fs_all[b]
        d = tc1(fc, fs)
        cv = _sc_topcand(d)
        outs.append(tc2(fc, fs, d, cv, W1, b1r, W2, b2r))

    out = jnp.stack(outs, axis=0)
    return jnp.transpose(out, (0, 2, 1)).reshape(B, C, P, P, M1, M2)


# speedup vs baseline: 1.0085x; 1.0085x over previous
import jax
import jax.numpy as jnp
from jax import lax
from jax.experimental import pallas as pl
from jax.experimental.pallas import tpu as pltpu
from jax.experimental.pallas import tpu_sc as plsc

_K = 5
_L = 16
_NCAND = _K * _L


def _tc_d_body(fc_ref, fs_ref, d_ref):
    f32 = jnp.float32
    fc = fc_ref[...]
    fs = fs_ref[...]
    m, F = fc.shape
    eps = f32(1e-8)

    hi = jax.lax.Precision.HIGHEST
    ones_row_F = jnp.ones((1, F), f32)
    nc_col = jnp.sum(fc * fc, axis=1, keepdims=True)
    nc_row = jax.lax.dot_general(ones_row_F, fc * fc, (((1,), (1,)), ((), ())),
                                 precision=hi, preferred_element_type=f32)
    ns_row = jax.lax.dot_general(ones_row_F, fs * fs, (((1,), (1,)), ((), ())),
                                 precision=hi, preferred_element_type=f32)

    def ncc(x, x_norm_row):
        g = jax.lax.dot_general(fc, x, (((1,), (1,)), ((), ())),
                                preferred_element_type=f32)
        return (g + eps) / (jnp.sqrt(nc_col * x_norm_row) + eps)

    d_ref[pl.ds(0, m), :] = ncc(fc, nc_row)
    d_ref[pl.ds(m, m), :] = ncc(fs, ns_row)


def _sc_topcand_body(d_hbm, cv_hbm, buf0, buf1, oval, sem0, sem1):
    info = plsc.get_sparse_core_info()
    nw = info.num_cores * info.num_subcores
    wid = lax.axis_index("s") * info.num_cores + lax.axis_index("c")
    ch, n = buf0.shape
    rows = cv_hbm.shape[0]
    rpw = rows // nw
    n_chunks = rpw // ch
    base = wid * rpw
    bufs = [buf0, buf1]
    sems = [sem0, sem1]
    neg_inf = jnp.full((_L,), -jnp.inf, jnp.float32)

    h = pltpu.async_copy(d_hbm.at[pl.ds(base, ch)], bufs[0], sems[0])
    for c in range(n_chunks):
        nxt = None
        if c + 1 < n_chunks:
            nxt = pltpu.async_copy(
                d_hbm.at[pl.ds(base + (c + 1) * ch, ch)],
                bufs[(c + 1) % 2], sems[(c + 1) % 2])
        h.wait()
        buf = bufs[c % 2]

        def row_body(r, _, buf=buf):
            v = [neg_inf] * _K
            for blk in range(n // _L):
                x = buf[r, pl.ds(blk * _L, _L)]
                for t in range(_K):
                    lo = jnp.minimum(x, v[t])
                    v[t] = jnp.maximum(x, v[t])
                    x = lo
            for t in range(_K):
                oval[r, pl.ds(t * _L, _L)] = v[t]
            return 0

        lax.fori_loop(0, ch, row_body, 0)
        pltpu.sync_copy(oval, cv_hbm.at[pl.ds(base + c * ch, ch)])
        h = nxt


def _sc_topcand(d_flat):
    rows, n = d_flat.shape
    ch = 16
    mesh = plsc.VectorSubcoreMesh(core_axis_name="c", subcore_axis_name="s")
    fn = pl.kernel(
        _sc_topcand_body,
        out_type=jax.ShapeDtypeStruct((rows, _NCAND), jnp.float32),
        mesh=mesh,
        scratch_types=[
            pltpu.VMEM((ch, n), jnp.float32),
            pltpu.VMEM((ch, n), jnp.float32),
            pltpu.VMEM((ch, _NCAND), jnp.float32),
            pltpu.SemaphoreType.DMA,
            pltpu.SemaphoreType.DMA,
        ],
        cost_estimate=pl.CostEstimate(
            flops=rows * n * 2 * _K,
            bytes_accessed=rows * n * 4 + rows * _NCAND * 4,
            transcendentals=0,
        ),
    )
    return fn(d_flat)


def _tc_agg_body(fc_ref, fs_ref, d_ref, cv_ref, W1_ref, b1_ref, W2_ref,
                 b2_ref, out_ref):
    f32 = jnp.float32
    fc = fc_ref[...]
    fs = fs_ref[...]
    m, F = fc.shape
    n = fs.shape[0]
    D = d_ref[...]
    val = cv_ref[...]

    lane80 = lax.broadcasted_iota(jnp.int32, val.shape, 1)
    big = jnp.int32(1 << 30)
    mx = jnp.max(val, axis=1, keepdims=True)
    for _ in range(_K - 1):
        am = jnp.min(jnp.where(val == mx, lane80, big), axis=1, keepdims=True)
        val = jnp.where(lane80 == am, -jnp.inf, val)
        mx = jnp.max(val, axis=1, keepdims=True)
    T = mx

    bf16 = jnp.bfloat16
    sel_gt = (D > T).astype(bf16)
    eq = (D == T).astype(bf16)
    ones_col_n = jnp.ones((n, 1), f32)

    def mm(a, b):
        return jax.lax.dot_general(a, b, (((1,), (0,)), ((), ())),
                                   preferred_element_type=f32)

    g = mm(sel_gt, ones_col_n.astype(bf16))
    r_iota = lax.broadcasted_iota(jnp.int32, (n, n), 0)
    c_iota = lax.broadcasted_iota(jnp.int32, (n, n), 1)
    U = (r_iota <= c_iota).astype(bf16)
    csum = mm(eq, U)
    S = sel_gt.astype(f32) + eq.astype(f32) * (
        csum <= (jnp.float32(_K) - g)).astype(f32)

    S2 = S[:m, :]
    S1 = S[m:, :]

    ones_row_m = jnp.ones((1, m), f32)
    cnt2 = jax.lax.dot_general(ones_row_m, S2, (((1,), (0,)), ((), ())),
                               preferred_element_type=f32)
    cnt1 = jax.lax.dot_general(ones_row_m, S1, (((1,), (0,)), ((), ())),
                               preferred_element_type=f32)
    S2w = S2 * jax.lax.rsqrt(jnp.maximum(cnt2, 1.0))
    S1w = S1 * jax.lax.rsqrt(jnp.maximum(cnt1, 1.0))

    c_in = f32((2.0 * _K) ** -0.5)
    W1 = W1_ref[...]
    W2 = W2_ref[...]
    b1 = b1_ref[...]
    b2 = b2_ref[...]

    agg1 = (mm(S2w, fc) + mm(S1w, fs)) * c_in
    h1 = jnp.maximum(mm(agg1, W1) + b1, 0.0)
    h1s = jnp.maximum(b1, 0.0)

    rs1 = mm(S1w, ones_col_n)
    agg2 = (mm(S2w, h1) + rs1 * h1s) * c_in
    out_ref[...] = mm(agg2, W2) + b2


def kernel(ys, yc, W1, b1, W2, b2):
    B, N1, N2, F = ys.shape
    _, C, P, _, M1, M2 = yc.shape
    n = N1 * N2
    m = M1 * M2
    fs_all = ys.reshape(B, n, F)
    fc_all = jnp.transpose(yc, (0, 4, 5, 1, 2, 3)).reshape(B, m, F)
    b1r = b1.reshape(1, F)
    b2r = b2.reshape(1, F)

    tc1 = pl.pallas_call(
        _tc_d_body,
        out_shape=jax.ShapeDtypeStruct((2 * m, n), jnp.float32),
    )
    tc2 = pl.pallas_call(
        _tc_agg_body,
        out_shape=jax.ShapeDtypeStruct((m, F), jnp.float32),
    )

    ds = [tc1(fc_all[b], fs_all[b]) for b in range(B)]
    cvs = [_sc_topcand(d) for d in ds]
    outs = [tc2(fc_all[b], fs_all[b], ds[b], cvs[b], W1, b1r, W2, b2r)
            for b in range(B)]

    out = jnp.stack(outs, axis=0)
    return jnp.transpose(out, (0, 2, 1)).reshape(B, C, P, P, M1, M2)


# speedup vs baseline: 1.2895x; 1.2787x over previous
import jax
import jax.numpy as jnp
from jax import lax
from jax.experimental import pallas as pl
from jax.experimental.pallas import tpu as pltpu
from jax.experimental.pallas import tpu_sc as plsc

_K = 5
_L = 16
_NCAND = _K * _L


def _tc_d_body(fc_ref, fs_ref, d_ref):
    f32 = jnp.float32
    fc = fc_ref[0]
    fs = fs_ref[0]
    m, F = fc.shape
    eps = f32(1e-8)

    hi = jax.lax.Precision.HIGHEST
    ones_row_F = jnp.ones((1, F), f32)
    nc_col = jnp.sum(fc * fc, axis=1, keepdims=True)
    nc_row = jax.lax.dot_general(ones_row_F, fc * fc, (((1,), (1,)), ((), ())),
                                 precision=hi, preferred_element_type=f32)
    ns_row = jax.lax.dot_general(ones_row_F, fs * fs, (((1,), (1,)), ((), ())),
                                 precision=hi, preferred_element_type=f32)

    def ncc(x, x_norm_row):
        g = jax.lax.dot_general(fc, x, (((1,), (1,)), ((), ())),
                                preferred_element_type=f32)
        return (g + eps) / (jnp.sqrt(nc_col * x_norm_row) + eps)

    d_ref[0, pl.ds(0, m), :] = ncc(fc, nc_row)
    d_ref[0, pl.ds(m, m), :] = ncc(fs, ns_row)


def _sc_topcand_body(d_hbm, cv_hbm, buf0, buf1, oval, sem0, sem1):
    info = plsc.get_sparse_core_info()
    nw = info.num_cores * info.num_subcores
    wid = lax.axis_index("s") * info.num_cores + lax.axis_index("c")
    ch, n = buf0.shape
    rows = cv_hbm.shape[0]
    rpw = rows // nw
    n_chunks = rpw // ch
    base = wid * rpw
    bufs = [buf0, buf1]
    sems = [sem0, sem1]
    neg_inf = jnp.full((_L,), -jnp.inf, jnp.float32)

    h = pltpu.async_copy(d_hbm.at[pl.ds(base, ch)], bufs[0], sems[0])
    for c in range(n_chunks):
        nxt = None
        if c + 1 < n_chunks:
            nxt = pltpu.async_copy(
                d_hbm.at[pl.ds(base + (c + 1) * ch, ch)],
                bufs[(c + 1) % 2], sems[(c + 1) % 2])
        h.wait()
        buf = bufs[c % 2]

        def row_body(r, _, buf=buf):
            v = [neg_inf] * _K
            for blk in range(n // _L):
                x = buf[r, pl.ds(blk * _L, _L)]
                for t in range(_K):
                    lo = jnp.minimum(x, v[t])
                    v[t] = jnp.maximum(x, v[t])
                    x = lo
            for t in range(_K):
                oval[r, pl.ds(t * _L, _L)] = v[t]
            return 0

        lax.fori_loop(0, ch, row_body, 0)
        pltpu.sync_copy(oval, cv_hbm.at[pl.ds(base + c * ch, ch)])
        h = nxt


def _sc_topcand(d_flat):
    rows, n = d_flat.shape
    ch = 16
    mesh = plsc.VectorSubcoreMesh(core_axis_name="c", subcore_axis_name="s")
    fn = pl.kernel(
        _sc_topcand_body,
        out_type=jax.ShapeDtypeStruct((rows, _NCAND), jnp.float32),
        mesh=mesh,
        scratch_types=[
            pltpu.VMEM((ch, n), jnp.float32),
            pltpu.VMEM((ch, n), jnp.float32),
            pltpu.VMEM((ch, _NCAND), jnp.float32),
            pltpu.SemaphoreType.DMA,
            pltpu.SemaphoreType.DMA,
        ],
        cost_estimate=pl.CostEstimate(
            flops=rows * n * 2 * _K,
            bytes_accessed=rows * n * 4 + rows * _NCAND * 4,
            transcendentals=0,
        ),
    )
    return fn(d_flat)


def _tc_agg_body(fc_ref, fs_ref, d_ref, cv_ref, W1_ref, b1_ref, W2_ref,
                 b2_ref, out_ref):
    f32 = jnp.float32
    fc = fc_ref[0]
    fs = fs_ref[0]
    m, F = fc.shape
    n = fs.shape[0]
    D = d_ref[0]
    val = cv_ref[0]

    lane80 = lax.broadcasted_iota(jnp.int32, val.shape, 1)
    big = jnp.int32(1 << 30)
    mx = jnp.max(val, axis=1, keepdims=True)
    for _ in range(_K - 1):
        am = jnp.min(jnp.where(val == mx, lane80, big), axis=1, keepdims=True)
        val = jnp.where(lane80 == am, -jnp.inf, val)
        mx = jnp.max(val, axis=1, keepdims=True)
    T = mx

    bf16 = jnp.bfloat16
    sel_gt = (D > T).astype(bf16)
    eq = (D == T).astype(bf16)
    ones_col_n = jnp.ones((n, 1), f32)

    def mm(a, b):
        return jax.lax.dot_general(a, b, (((1,), (0,)), ((), ())),
                                   preferred_element_type=f32)

    g = mm(sel_gt, ones_col_n.astype(bf16))
    r_iota = lax.broadcasted_iota(jnp.int32, (n, n), 0)
    c_iota = lax.broadcasted_iota(jnp.int32, (n, n), 1)
    U = (r_iota <= c_iota).astype(bf16)
    csum = mm(eq, U)
    S = sel_gt.astype(f32) + eq.astype(f32) * (
        csum <= (jnp.float32(_K) - g)).astype(f32)

    S2 = S[:m, :]
    S1 = S[m:, :]

    ones_row_m = jnp.ones((1, m), f32)
    cnt2 = jax.lax.dot_general(ones_row_m, S2, (((1,), (0,)), ((), ())),
                               preferred_element_type=f32)
    cnt1 = jax.lax.dot_general(ones_row_m, S1, (((1,), (0,)), ((), ())),
                               preferred_element_type=f32)
    S2w = S2 * jax.lax.rsqrt(jnp.maximum(cnt2, 1.0))
    S1w = S1 * jax.lax.rsqrt(jnp.maximum(cnt1, 1.0))

    c_in = f32((2.0 * _K) ** -0.5)
    W1 = W1_ref[...]
    W2 = W2_ref[...]
    b1 = b1_ref[...]
    b2 = b2_ref[...]

    agg1 = (mm(S2w, fc) + mm(S1w, fs)) * c_in
    h1 = jnp.maximum(mm(agg1, W1) + b1, 0.0)
    h1s = jnp.maximum(b1, 0.0)

    rs1 = mm(S1w, ones_col_n)
    agg2 = (mm(S2w, h1) + rs1 * h1s) * c_in
    out_ref[0] = mm(agg2, W2) + b2


def kernel(ys, yc, W1, b1, W2, b2):
    B, N1, N2, F = ys.shape
    _, C, P, _, M1, M2 = yc.shape
    n = N1 * N2
    m = M1 * M2
    fs_all = ys.reshape(B, n, F)
    fc_all = jnp.transpose(yc, (0, 4, 5, 1, 2, 3)).reshape(B, m, F)
    b1r = b1.reshape(1, F)
    b2r = b2.reshape(1, F)

    tc1 = pl.pallas_call(
        _tc_d_body,
        grid=(B,),
        in_specs=[
            pl.BlockSpec((1, m, F), lambda b: (b, 0, 0)),
            pl.BlockSpec((1, n, F), lambda b: (b, 0, 0)),
        ],
        out_specs=pl.BlockSpec((1, 2 * m, n), lambda b: (b, 0, 0)),
        out_shape=jax.ShapeDtypeStruct((B, 2 * m, n), jnp.float32),
    )
    tc2 = pl.pallas_call(
        _tc_agg_body,
        grid=(B,),
        in_specs=[
            pl.BlockSpec((1, m, F), lambda b: (b, 0, 0)),
            pl.BlockSpec((1, n, F), lambda b: (b, 0, 0)),
            pl.BlockSpec((1, 2 * m, n), lambda b: (b, 0, 0)),
            pl.BlockSpec((1, 2 * m, _NCAND), lambda b: (b, 0, 0)),
            pl.BlockSpec((F, F), lambda b: (0, 0)),
            pl.BlockSpec((1, F), lambda b: (0, 0)),
            pl.BlockSpec((F, F), lambda b: (0, 0)),
            pl.BlockSpec((1, F), lambda b: (0, 0)),
        ],
        out_specs=pl.BlockSpec((1, m, F), lambda b: (b, 0, 0)),
        out_shape=jax.ShapeDtypeStruct((B, m, F), jnp.float32),
    )

    d_all = tc1(fc_all, fs_all)
    cv = _sc_topcand(d_all.reshape(B * 2 * m, n))
    out = tc2(fc_all, fs_all, d_all, cv.reshape(B, 2 * m, _NCAND),
              W1, b1r, W2, b2r)
    return jnp.transpose(out, (0, 2, 1)).reshape(B, C, P, P, M1, M2)
